# baseline (speedup 1.0000x reference)
import numpy as np
import jax
import jax.numpy as jnp
from jax import lax
from jax.experimental import pallas as pl
from jax.experimental.pallas import tpu as pltpu

_S = 2048
_DMODEL = 2048
_NH = 16
_DH = 128
_TOPK = 512
_NEG = np.float32(-1e30)


def _hadamard(d):
    m = np.array([[1.0]], dtype=np.float32)
    while m.shape[0] < d:
        m = np.block([[m, m], [m, -m]]).astype(np.float32)
    return m


def _mm(a, b, prec=lax.Precision.DEFAULT):
    return lax.dot_general(a, b, (((1,), (0,)), ((), ())),
                           precision=prec,
                           preferred_element_type=jnp.float32)


def _roll(x, shift, axis):
    return pltpu.roll(x, shift, axis)


def _matmul_body(x_ref, wq_ref, wk_ref, ww_ref, q2_ref, k2_ref, w_ref):
    x = x_ref[...]
    q2_ref[...] = _mm(x, wq_ref[...])
    k2_ref[...] = _mm(x, wk_ref[...])
    w_ref[...] = _mm(x, ww_ref[...]) * jnp.float32(_NH ** -0.5)


def _rot_body(q2_ref, k2_ref, hm_ref, q_ref, k_ref):
    hm = hm_ref[...]
    hs = jnp.float32(_DH ** -0.5)
    for h in range(_NH):
        sl = slice(h * _DH, (h + 1) * _DH)
        q_ref[:, sl] = _mm(q2_ref[:, sl], hm) * hs
    k_ref[...] = _mm(k2_ref[...], hm) * hs


def _cmp_swap(v, ii, j, asc, col):
    W = v.shape[1]
    bit = (col & j) != 0
    vl = _roll(v, W - j, 1)
    vr = _roll(v, j, 1)
    il = _roll(ii, W - j, 1)
    ir = _roll(ii, j, 1)
    pv = jnp.where(bit, vr, vl)
    pi = jnp.where(bit, ir, il)
    beat = (v > pv) | ((v == pv) & (ii < pi))
    swap = (bit == asc) != beat
    return jnp.where(swap, pv, v), jnp.where(swap, pi, ii)


def _winner(av, ai, bv, bi):
    beat = (av > bv) | ((av == bv) & (ai < bi))
    return jnp.where(beat, av, bv), jnp.where(beat, ai, bi)


def _topk_desc(scores, col):
    v, ii = scores, col
    kk = 2
    while kk <= _TOPK:
        asc = (col & kk) == 0
        j = kk // 2
        while j >= 1:
            v, ii = _cmp_swap(v, ii, j, asc, col)
            j //= 2
        kk *= 2
    av = jnp.concatenate([v[:, 0:512], v[:, 1024:1536]], 1)
    ai = jnp.concatenate([ii[:, 0:512], ii[:, 1024:1536]], 1)
    bv = jnp.concatenate([v[:, 512:1024], v[:, 1536:2048]], 1)
    bi = jnp.concatenate([ii[:, 512:1024], ii[:, 1536:2048]], 1)
    v, ii = _winner(av, ai, bv, bi)
    col2 = lax.broadcasted_iota(jnp.int32, v.shape, 1)
    asc2 = (col2 & 512) == 0
    j = 256
    while j >= 1:
        v, ii = _cmp_swap(v, ii, j, asc2, col2)
        j //= 2
    v, ii = _winner(v[:, 0:512], ii[:, 0:512], v[:, 512:1024], ii[:, 512:1024])
    col3 = lax.broadcasted_iota(jnp.int32, v.shape, 1)
    asc3 = col3 < 0
    j = 256
    while j >= 1:
        v, ii = _cmp_swap(v, ii, j, asc3, col3)
        j //= 2
    return v, ii


def _score_body(q_ref, kr_ref, w_ref, pos_ref, vals_ref, idx_ref):
    R = q_ref.shape[0]
    def _rb(x):
        u = lax.bitcast_convert_type(x, jnp.uint32)
        u = (u + jnp.uint32(0x8000)) & jnp.uint32(0xFFFF0000)
        return lax.bitcast_convert_type(u, jnp.float32)

    kr = kr_ref[...]
    w = w_ref[...]
    wb = _rb(w)
    acc = jnp.zeros((R, _S), jnp.float32)
    for h in range(_NH):
        qh = q_ref[:, h * _DH:(h + 1) * _DH]
        lg = lax.dot_general(qh, kr, (((1,), (1,)), ((), ())),
                             precision=lax.Precision.DEFAULT,
                             preferred_element_type=jnp.float32)
        lgb = _rb(jnp.maximum(lg, jnp.float32(0.0)))
        acc = acc + lgb * wb[:, h:h + 1]
    scores = acc * jnp.float32(_DH ** -0.5)
    col = lax.broadcasted_iota(jnp.int32, (R, _S), 1)
    pos = pos_ref[:, 0:1]
    scores = jnp.where(col <= pos, scores, _NEG)
    v, ii = _topk_desc(scores, col)
    vals_ref[...] = v
    idx_ref[...] = ii


def _project(x, wq, wk, ww):
    hm = jnp.asarray(_hadamard(_DH))
    q2, k2, wts = pl.pallas_call(
        _matmul_body,
        grid=(1,),
        in_specs=[
            pl.BlockSpec((_S, _DMODEL), lambda i: (0, 0)),
            pl.BlockSpec((_DMODEL, _NH * _DH), lambda i: (0, 0)),
            pl.BlockSpec((_DMODEL, _DH), lambda i: (0, 0)),
            pl.BlockSpec((_DMODEL, _NH), lambda i: (0, 0)),
        ],
        out_specs=[
            pl.BlockSpec((_S, _NH * _DH), lambda i: (0, 0)),
            pl.BlockSpec((_S, _DH), lambda i: (0, 0)),
            pl.BlockSpec((_S, _NH), lambda i: (0, 0)),
        ],
        out_shape=[
            jax.ShapeDtypeStruct((_S, _NH * _DH), jnp.float32),
            jax.ShapeDtypeStruct((_S, _DH), jnp.float32),
            jax.ShapeDtypeStruct((_S, _NH), jnp.float32),
        ],
    )(x, wq, wk, ww)
    q_rot, k_rot = pl.pallas_call(
        _rot_body,
        grid=(1,),
        in_specs=[
            pl.BlockSpec((_S, _NH * _DH), lambda i: (0, 0)),
            pl.BlockSpec((_S, _DH), lambda i: (0, 0)),
            pl.BlockSpec((_DH, _DH), lambda i: (0, 0)),
        ],
        out_specs=[
            pl.BlockSpec((_S, _NH * _DH), lambda i: (0, 0)),
            pl.BlockSpec((_S, _DH), lambda i: (0, 0)),
        ],
        out_shape=[
            jax.ShapeDtypeStruct((_S, _NH * _DH), jnp.float32),
            jax.ShapeDtypeStruct((_S, _DH), jnp.float32),
        ],
    )(q2, k2, hm)
    return q_rot, k_rot, wts


def _score_topk(q_rot, k_rot, wts, positions):
    pos2d = jnp.broadcast_to(positions.astype(jnp.int32)[:, None], (_S, 128))
    rb = 256
    return pl.pallas_call(
        _score_body,
        grid=(_S // rb,),
        in_specs=[
            pl.BlockSpec((rb, _NH * _DH), lambda i: (i, 0)),
            pl.BlockSpec((_S, _DH), lambda i: (0, 0)),
            pl.BlockSpec((rb, _NH), lambda i: (i, 0)),
            pl.BlockSpec((rb, 128), lambda i: (i, 0)),
        ],
        out_specs=[
            pl.BlockSpec((rb, _TOPK), lambda i: (i, 0)),
            pl.BlockSpec((rb, _TOPK), lambda i: (i, 0)),
        ],
        out_shape=[
            jax.ShapeDtypeStruct((_S, _TOPK), jnp.float32),
            jax.ShapeDtypeStruct((_S, _TOPK), jnp.int32),
        ],
    )(q_rot, k_rot, wts, pos2d)


def kernel(x, wq, wk, ww, positions):
    q_rot, k_rot, wts = _project(x, wq, wk, ww)
    return _score_topk(q_rot, k_rot, wts, positions)
